# baseline (speedup 1.0000x reference)
import jax
import jax.numpy as jnp
from jax.experimental import pallas as pl

_N = 1024
_M = 64
_NTX = 32
_NSYM = 8
_NSAMP = 16
_NB = 64
_C = 1.702


def _body(lq_ref, g_ref, G_ref, nv_ref, w_ref, out_ref):
    f32 = jnp.float32
    E = (jax.lax.broadcasted_iota(jnp.int32, (_NSAMP, 128), 0)
         == jax.lax.broadcasted_iota(jnp.int32, (_NSAMP, 128), 1) // _NSYM
         ).astype(f32)
    P = (jax.lax.broadcasted_iota(jnp.int32, (128, _NSYM), 0) % _NSYM
         == jax.lax.broadcasted_iota(jnp.int32, (128, _NSYM), 1)).astype(f32)
    srow = 2.0 * (jax.lax.broadcasted_iota(jnp.int32, (1, 128), 1)
                  % _NSYM).astype(f32) - 7.0
    iota8 = jax.lax.broadcasted_iota(jnp.int32, (_NTX, _NSAMP, _NSYM), 2)
    wv = w_ref[:, :]

    def body(b, carry):
        nv = jnp.maximum(0.01, nv_ref[pl.ds(b, 1), :])
        rho = 1.0 / nv
        s2r = jnp.sqrt(2.0 * rho)
        alpha = jnp.minimum(1.0, wv * nv)
        lq = lq_ref[b]
        g3 = g_ref[b]
        z3 = g3 + lq[:, None, :]
        mx = jnp.max(z3, axis=2, keepdims=True)
        cand = jnp.where(z3 == mx, iota8, 8)
        imin = jnp.min(cand, axis=2)
        s = 2.0 * imin.astype(f32) - 7.0
        S_rep = jnp.dot(s, E, preferred_element_type=f32)
        D = srow - S_rep
        csG = (_C * s2r) * G_ref[b]
        cB = jnp.dot(csG, S_rep, preferred_element_type=f32,
                     precision=jax.lax.Precision.HIGHEST)
        rows = []
        for xi in range(_NTX):
            ct = cB + csG[:, xi:xi + 1] * D[xi:xi + 1, :]
            mn = jnp.minimum(ct, 0.0)
            p = 1.0 + jnp.exp(-jnp.abs(ct))
            ssum = jnp.sum(mn, axis=0, keepdims=True)
            pr = p
            for h in (32, 16, 8, 4, 2, 1):
                pr = pr[:h] * pr[h:2 * h]
            rows.append(ssum - jnp.log(pr))
        Lcat = jnp.concatenate(rows, axis=0)
        ex = jnp.dot(Lcat, P, preferred_element_type=f32,
                     precision=jax.lax.Precision.HIGHEST) * (1.0 / _NSAMP)
        out = (1.0 - alpha) * lq + alpha * ex
        out = out - jnp.max(out, axis=1, keepdims=True)
        out_ref[b] = out
        return carry

    jax.lax.fori_loop(0, _NB, body, 0)


def kernel(log_qi, G, sqrt_2rho, n_var, w):
    del sqrt_2rho
    g = jax.random.gumbel(jax.random.key(42), (_NSAMP, _N, _NTX, _NSYM),
                          jnp.float32)
    g4 = jnp.transpose(g, (1, 2, 0, 3))
    nv = n_var.reshape(_N, 1)
    wv = jnp.asarray(w, jnp.float32).reshape(1, 1)
    return pl.pallas_call(
        _body,
        grid=(_N // _NB,),
        in_specs=[
            pl.BlockSpec((_NB, _NTX, _NSYM), lambda i: (i, 0, 0)),
            pl.BlockSpec((_NB, _NTX, _NSAMP, _NSYM), lambda i: (i, 0, 0, 0)),
            pl.BlockSpec((_NB, _M, _NTX), lambda i: (i, 0, 0)),
            pl.BlockSpec((_NB, 1), lambda i: (i, 0)),
            pl.BlockSpec((1, 1), lambda i: (0, 0)),
        ],
        out_specs=pl.BlockSpec((_NB, _NTX, _NSYM), lambda i: (i, 0, 0)),
        out_shape=jax.ShapeDtypeStruct((_N, _NTX, _NSYM), jnp.float32),
    )(log_qi, g4, G, nv, wv)
